# Initial kernel scaffold; baseline (speedup 1.0000x reference)
import jax
import jax.numpy as jnp
from jax.experimental import pallas as pl


def kernel(indices, x):
    raise NotImplementedError("write your pallas kernel here")


# speedup vs baseline: 1.2713x; 1.2713x over previous
import functools

import jax
import jax.numpy as jnp
from jax import lax
from jax.experimental import pallas as pl
from jax.experimental.pallas import tpu as pltpu
from jax.experimental.pallas import tpu_sc as plsc

_B = 50000
_D = 128
_CHUNK = 80
_NCHUNKS = _B // _CHUNK


@functools.cache
def _build_gather():
    info = plsc.get_sparse_core_info()
    nc, ns = info.num_cores, info.num_subcores
    nw = nc * ns
    per = -(-_NCHUNKS // nw)

    mesh = plsc.VectorSubcoreMesh(core_axis_name="c", subcore_axis_name="s")

    @functools.partial(
        pl.kernel,
        mesh=mesh,
        out_type=jax.ShapeDtypeStruct((_B, _D), jnp.float32),
        scratch_types=[
            pltpu.VMEM((_CHUNK,), jnp.int32),
            pltpu.VMEM((_CHUNK, _D), jnp.float32),
            pltpu.SemaphoreType.DMA,
        ],
    )
    def gather_k(idx_hbm, table_hbm, out_hbm, idx_v, rows_v, sem):
        wid = lax.axis_index("s") * nc + lax.axis_index("c")

        def body(j, carry):
            t = wid + nw * j

            @pl.when(t < _NCHUNKS)
            def _():
                base = t * _CHUNK
                pltpu.sync_copy(idx_hbm.at[pl.ds(base, _CHUNK)], idx_v)
                pltpu.async_copy(table_hbm.at[idx_v], rows_v, sem).wait()
                pltpu.sync_copy(rows_v, out_hbm.at[pl.ds(base, _CHUNK)])

            return carry

        lax.fori_loop(0, per, body, None)

    return gather_k


def kernel(indices, x):
    return _build_gather()(indices.reshape(-1), x)


# speedup vs baseline: 1.6188x; 1.2734x over previous
import functools

import jax
import jax.numpy as jnp
from jax import lax
from jax.experimental import pallas as pl
from jax.experimental.pallas import tpu as pltpu
from jax.experimental.pallas import tpu_sc as plsc

_B = 50000
_D = 128
_CHUNK = 80
_NCHUNKS = _B // _CHUNK


@functools.cache
def _build_gather():
    info = plsc.get_sparse_core_info()
    nc, ns = info.num_cores, info.num_subcores
    nw = nc * ns
    nslot = -(-_NCHUNKS // nw)
    assert nslot % 2 == 0

    mesh = plsc.VectorSubcoreMesh(core_axis_name="c", subcore_axis_name="s")

    @functools.partial(
        pl.kernel,
        mesh=mesh,
        out_type=jax.ShapeDtypeStruct((_B, _D), jnp.float32),
        scratch_types=[
            pltpu.VMEM((2, _CHUNK), jnp.int32),
            pltpu.VMEM((2, _CHUNK, _D), jnp.float32),
            pltpu.SemaphoreType.DMA,
            pltpu.SemaphoreType.DMA,
            pltpu.SemaphoreType.DMA,
            pltpu.SemaphoreType.DMA,
            pltpu.SemaphoreType.DMA,
            pltpu.SemaphoreType.DMA,
        ],
    )
    def gather_k(idx_hbm, table_hbm, out_hbm, idx_v, rows_v,
                 isem0, isem1, gsem0, gsem1, ssem0, ssem1):
        wid = lax.axis_index("s") * nc + lax.axis_index("c")
        isem = (isem0, isem1)
        gsem = (gsem0, gsem1)
        ssem = (ssem0, ssem1)

        def chunk_base(s):
            return jnp.minimum(wid + nw * s, _NCHUNKS - 1) * _CHUNK

        for b in range(2):
            pltpu.async_copy(idx_hbm.at[pl.ds(chunk_base(b), _CHUNK)],
                             idx_v.at[b], isem[b])

        def body(j, carry):
            for b in range(2):
                s = 2 * j + b
                t = wid + nw * s

                @pl.when(j >= 1)
                def _():
                    pltpu.make_async_copy(
                        rows_v.at[b], out_hbm.at[pl.ds(0, _CHUNK)],
                        ssem[b]).wait()

                pltpu.make_async_copy(
                    idx_hbm.at[pl.ds(0, _CHUNK)], idx_v.at[b],
                    isem[b]).wait()

                pltpu.async_copy(table_hbm.at[idx_v.at[b]], rows_v.at[b],
                                 gsem[b]).wait()

                @pl.when(s + 2 < nslot)
                def _():
                    pltpu.async_copy(
                        idx_hbm.at[pl.ds(chunk_base(s + 2), _CHUNK)],
                        idx_v.at[b], isem[b])

                @pl.when(t < _NCHUNKS)
                def _():
                    pltpu.async_copy(
                        rows_v.at[b], out_hbm.at[pl.ds(t * _CHUNK, _CHUNK)],
                        ssem[b])
            return carry

        lax.fori_loop(0, nslot // 2, body, None)

        pltpu.make_async_copy(rows_v.at[0], out_hbm.at[pl.ds(0, _CHUNK)],
                              ssem[0]).wait()

        @pl.when(wid + nw * (nslot - 1) < _NCHUNKS)
        def _():
            pltpu.make_async_copy(rows_v.at[1], out_hbm.at[pl.ds(0, _CHUNK)],
                                  ssem[1]).wait()

    return gather_k


def kernel(indices, x):
    return _build_gather()(indices.reshape(-1), x)


# speedup vs baseline: 1.9550x; 1.2077x over previous
import functools

import jax
import jax.numpy as jnp
from jax import lax
from jax.experimental import pallas as pl
from jax.experimental.pallas import tpu as pltpu
from jax.experimental.pallas import tpu_sc as plsc

_B = 50000
_D = 128
_CHUNK = 80
_NCHUNKS = _B // _CHUNK
_NBUF = 4


@functools.cache
def _build_gather():
    info = plsc.get_sparse_core_info()
    nc, ns = info.num_cores, info.num_subcores
    nw = nc * ns
    nslot = -(-_NCHUNKS // nw)
    assert nslot % _NBUF == 0

    mesh = plsc.VectorSubcoreMesh(core_axis_name="c", subcore_axis_name="s")

    @functools.partial(
        pl.kernel,
        mesh=mesh,
        out_type=jax.ShapeDtypeStruct((_B, _D), jnp.float32),
        scratch_types=[
            pltpu.VMEM((_NBUF, _CHUNK), jnp.int32),
            pltpu.VMEM((_NBUF, _CHUNK, _D), jnp.float32),
        ] + [pltpu.SemaphoreType.DMA] * (3 * _NBUF),
    )
    def gather_k(idx_hbm, table_hbm, out_hbm, idx_v, rows_v, *sems):
        isem = sems[0:_NBUF]
        gsem = sems[_NBUF:2 * _NBUF]
        ssem = sems[2 * _NBUF:3 * _NBUF]
        wid = lax.axis_index("s") * nc + lax.axis_index("c")

        def chunk_base(s):
            return jnp.minimum(wid + nw * s, _NCHUNKS - 1) * _CHUNK

        def start_idx(s, b):
            pltpu.async_copy(idx_hbm.at[pl.ds(chunk_base(s), _CHUNK)],
                             idx_v.at[b], isem[b])

        def wait_idx(b):
            pltpu.make_async_copy(idx_hbm.at[pl.ds(0, _CHUNK)],
                                  idx_v.at[b], isem[b]).wait()

        def start_gather(b):
            pltpu.async_copy(table_hbm.at[idx_v.at[b]], rows_v.at[b],
                             gsem[b])

        def wait_gather(b):
            pltpu.make_async_copy(table_hbm.at[idx_v.at[b]], rows_v.at[b],
                                  gsem[b]).wait()

        def start_store(s, b):
            t = wid + nw * s

            @pl.when(t < _NCHUNKS)
            def _():
                pltpu.async_copy(rows_v.at[b],
                                 out_hbm.at[pl.ds(t * _CHUNK, _CHUNK)],
                                 ssem[b])

        def wait_store(b):
            pltpu.make_async_copy(rows_v.at[b], out_hbm.at[pl.ds(0, _CHUNK)],
                                  ssem[b]).wait()

        for b in range(_NBUF):
            start_idx(b, b)
        for b in range(2):
            wait_idx(b)
            start_gather(b)

        def body(j, carry):
            for b in range(_NBUF):
                s = _NBUF * j + b

                wait_gather(b)
                start_store(s, b)

                b2 = (b + 2) % _NBUF

                @pl.when(s + 2 < nslot)
                def _():
                    @pl.when(s >= 2)
                    def _():
                        wait_store(b2)
                    wait_idx(b2)
                    start_gather(b2)

                @pl.when(s + _NBUF < nslot)
                def _():
                    start_idx(s + _NBUF, b)
            return carry

        lax.fori_loop(0, nslot // _NBUF, body, None)

        for b in range(_NBUF - 1):
            wait_store(b)

        @pl.when(wid + nw * (nslot - 1) < _NCHUNKS)
        def _():
            wait_store(_NBUF - 1)

    return gather_k


def kernel(indices, x):
    return _build_gather()(indices.reshape(-1), x)


# speedup vs baseline: 1.9937x; 1.0198x over previous
import functools

import jax
import jax.numpy as jnp
from jax import lax
from jax.experimental import pallas as pl
from jax.experimental.pallas import tpu as pltpu
from jax.experimental.pallas import tpu_sc as plsc

_B = 50000
_D = 128
_CHUNK = 80
_NCHUNKS = _B // _CHUNK
_NBUF = 5
_GDEPTH = 3


@functools.cache
def _build_gather():
    info = plsc.get_sparse_core_info()
    nc, ns = info.num_cores, info.num_subcores
    nw = nc * ns
    nslot = -(-_NCHUNKS // nw)
    assert nslot % _NBUF == 0

    mesh = plsc.VectorSubcoreMesh(core_axis_name="c", subcore_axis_name="s")

    @functools.partial(
        pl.kernel,
        mesh=mesh,
        out_type=jax.ShapeDtypeStruct((_B, _D), jnp.float32),
        scratch_types=[
            pltpu.VMEM((_NBUF, _CHUNK), jnp.int32),
            pltpu.VMEM((_NBUF, _CHUNK, _D), jnp.float32),
        ] + [pltpu.SemaphoreType.DMA] * (3 * _NBUF),
    )
    def gather_k(idx_hbm, table_hbm, out_hbm, idx_v, rows_v, *sems):
        isem = sems[0:_NBUF]
        gsem = sems[_NBUF:2 * _NBUF]
        ssem = sems[2 * _NBUF:3 * _NBUF]
        wid = lax.axis_index("s") * nc + lax.axis_index("c")

        def chunk_base(s):
            return jnp.minimum(wid + nw * s, _NCHUNKS - 1) * _CHUNK

        def start_idx(s, b):
            pltpu.async_copy(idx_hbm.at[pl.ds(chunk_base(s), _CHUNK)],
                             idx_v.at[b], isem[b])

        def wait_idx(b):
            pltpu.make_async_copy(idx_hbm.at[pl.ds(0, _CHUNK)],
                                  idx_v.at[b], isem[b]).wait()

        def start_gather(b):
            pltpu.async_copy(table_hbm.at[idx_v.at[b]], rows_v.at[b],
                             gsem[b])

        def wait_gather(b):
            pltpu.make_async_copy(table_hbm.at[idx_v.at[b]], rows_v.at[b],
                                  gsem[b]).wait()

        def start_store(s, b):
            t = wid + nw * s

            @pl.when(t < _NCHUNKS)
            def _():
                pltpu.async_copy(rows_v.at[b],
                                 out_hbm.at[pl.ds(t * _CHUNK, _CHUNK)],
                                 ssem[b])

        def wait_store(b):
            pltpu.make_async_copy(rows_v.at[b], out_hbm.at[pl.ds(0, _CHUNK)],
                                  ssem[b]).wait()

        for b in range(_NBUF):
            start_idx(b, b)
        for b in range(_GDEPTH):
            wait_idx(b)
            start_gather(b)

        def body(j, carry):
            for b in range(_NBUF):
                s = _NBUF * j + b

                wait_gather(b)
                start_store(s, b)

                b2 = (b + _GDEPTH) % _NBUF

                @pl.when(s + _GDEPTH < nslot)
                def _():
                    @pl.when(s >= _NBUF - _GDEPTH)
                    def _():
                        wait_store(b2)
                    wait_idx(b2)
                    start_gather(b2)

                @pl.when(s + _NBUF < nslot)
                def _():
                    start_idx(s + _NBUF, b)
            return carry

        lax.fori_loop(0, nslot // _NBUF, body, None)

        for b in range(_NBUF - 1):
            wait_store(b)

        @pl.when(wid + nw * (nslot - 1) < _NCHUNKS)
        def _():
            wait_store(_NBUF - 1)

    return gather_k


def kernel(indices, x):
    return _build_gather()(indices.reshape(-1), x)


# speedup vs baseline: 2.0951x; 1.0508x over previous
import functools

import jax
import jax.numpy as jnp
from jax import lax
from jax.experimental import pallas as pl
from jax.experimental.pallas import tpu as pltpu
from jax.experimental.pallas import tpu_sc as plsc

_B = 50000
_D = 128
_CHUNK = 112
_NBUF = 7
_GDEPTH = 4


@functools.cache
def _build_gather():
    info = plsc.get_sparse_core_info()
    nc, ns = info.num_cores, info.num_subcores
    nw = nc * ns
    nslot = -(-_B // (nw * _CHUNK))
    assert nslot % _NBUF == 0

    mesh = plsc.VectorSubcoreMesh(core_axis_name="c", subcore_axis_name="s")

    @functools.partial(
        pl.kernel,
        mesh=mesh,
        out_type=jax.ShapeDtypeStruct((_B, _D), jnp.float32),
        scratch_types=[
            pltpu.VMEM((_NBUF, _CHUNK), jnp.int32),
            pltpu.VMEM((_NBUF, _CHUNK, _D), jnp.float32),
        ] + [pltpu.SemaphoreType.DMA] * (3 * _NBUF),
    )
    def gather_k(idx_hbm, table_hbm, out_hbm, idx_v, rows_v, *sems):
        isem = sems[0:_NBUF]
        gsem = sems[_NBUF:2 * _NBUF]
        ssem = sems[2 * _NBUF:3 * _NBUF]
        wid = lax.axis_index("s") * nc + lax.axis_index("c")

        def chunk_base(s):
            return jnp.minimum((wid * nslot + s) * _CHUNK, _B - _CHUNK)

        def start_idx(s, b):
            pltpu.async_copy(idx_hbm.at[pl.ds(chunk_base(s), _CHUNK)],
                             idx_v.at[b], isem[b])

        def wait_idx(b):
            pltpu.make_async_copy(idx_hbm.at[pl.ds(0, _CHUNK)],
                                  idx_v.at[b], isem[b]).wait()

        def start_gather(b):
            pltpu.async_copy(table_hbm.at[idx_v.at[b]], rows_v.at[b],
                             gsem[b])

        def wait_gather(b):
            pltpu.make_async_copy(table_hbm.at[idx_v.at[b]], rows_v.at[b],
                                  gsem[b]).wait()

        def start_store(s, b):
            pltpu.async_copy(rows_v.at[b],
                             out_hbm.at[pl.ds(chunk_base(s), _CHUNK)],
                             ssem[b])

        def wait_store(b):
            pltpu.make_async_copy(rows_v.at[b], out_hbm.at[pl.ds(0, _CHUNK)],
                                  ssem[b]).wait()

        for b in range(_NBUF):
            start_idx(b, b)
        for b in range(_GDEPTH):
            wait_idx(b)
            start_gather(b)

        def body(j, carry):
            for b in range(_NBUF):
                s = _NBUF * j + b

                wait_gather(b)
                start_store(s, b)

                b2 = (b + _GDEPTH) % _NBUF

                @pl.when(s + _GDEPTH < nslot)
                def _():
                    @pl.when(s >= _NBUF - _GDEPTH)
                    def _():
                        wait_store(b2)
                    wait_idx(b2)
                    start_gather(b2)

                @pl.when(s + _NBUF < nslot)
                def _():
                    start_idx(s + _NBUF, b)
            return carry

        lax.fori_loop(0, nslot // _NBUF, body, None)

        for b in range(_NBUF):
            wait_store(b)

    return gather_k


def kernel(indices, x):
    return _build_gather()(indices.reshape(-1), x)


# speedup vs baseline: 2.1187x; 1.0113x over previous
import functools

import jax
import jax.numpy as jnp
from jax import lax
from jax.experimental import pallas as pl
from jax.experimental.pallas import tpu as pltpu
from jax.experimental.pallas import tpu_sc as plsc

_B = 50000
_D = 128
_CHUNK = 112
_NBUF = 7
_GDEPTH = 5


@functools.cache
def _build_gather():
    info = plsc.get_sparse_core_info()
    nc, ns = info.num_cores, info.num_subcores
    nw = nc * ns
    nslot = -(-_B // (nw * _CHUNK))
    assert nslot % _NBUF == 0

    mesh = plsc.VectorSubcoreMesh(core_axis_name="c", subcore_axis_name="s")

    @functools.partial(
        pl.kernel,
        mesh=mesh,
        out_type=jax.ShapeDtypeStruct((_B, _D), jnp.float32),
        scratch_types=[
            pltpu.VMEM((_NBUF, _CHUNK), jnp.int32),
            pltpu.VMEM((_NBUF, _CHUNK, _D), jnp.float32),
        ] + [pltpu.SemaphoreType.DMA] * (3 * _NBUF),
    )
    def gather_k(idx_hbm, table_hbm, out_hbm, idx_v, rows_v, *sems):
        isem = sems[0:_NBUF]
        gsem = sems[_NBUF:2 * _NBUF]
        ssem = sems[2 * _NBUF:3 * _NBUF]
        wid = lax.axis_index("s") * nc + lax.axis_index("c")

        def chunk_base(s):
            return jnp.minimum((wid * nslot + s) * _CHUNK, _B - _CHUNK)

        def start_idx(s, b):
            pltpu.async_copy(idx_hbm.at[pl.ds(chunk_base(s), _CHUNK)],
                             idx_v.at[b], isem[b])

        def wait_idx(b):
            pltpu.make_async_copy(idx_hbm.at[pl.ds(0, _CHUNK)],
                                  idx_v.at[b], isem[b]).wait()

        def start_gather(b):
            pltpu.async_copy(table_hbm.at[idx_v.at[b]], rows_v.at[b],
                             gsem[b])

        def wait_gather(b):
            pltpu.make_async_copy(table_hbm.at[idx_v.at[b]], rows_v.at[b],
                                  gsem[b]).wait()

        def start_store(s, b):
            pltpu.async_copy(rows_v.at[b],
                             out_hbm.at[pl.ds(chunk_base(s), _CHUNK)],
                             ssem[b])

        def wait_store(b):
            pltpu.make_async_copy(rows_v.at[b], out_hbm.at[pl.ds(0, _CHUNK)],
                                  ssem[b]).wait()

        for b in range(_NBUF):
            start_idx(b, b)
        for b in range(_GDEPTH):
            wait_idx(b)
            start_gather(b)

        def body(j, carry):
            for b in range(_NBUF):
                s = _NBUF * j + b

                wait_gather(b)
                start_store(s, b)

                b2 = (b + _GDEPTH) % _NBUF

                @pl.when(s + _GDEPTH < nslot)
                def _():
                    @pl.when(s >= _NBUF - _GDEPTH)
                    def _():
                        wait_store(b2)
                    wait_idx(b2)
                    start_gather(b2)

                @pl.when(s + _NBUF < nslot)
                def _():
                    start_idx(s + _NBUF, b)
            return carry

        lax.fori_loop(0, nslot // _NBUF, body, None)

        for b in range(_NBUF):
            wait_store(b)

    return gather_k


def kernel(indices, x):
    return _build_gather()(indices.reshape(-1), x)
